# baseline (speedup 1.0000x reference)
import dataclasses
import functools

import jax
import jax.numpy as jnp
from jax import lax
from jax.experimental import pallas as pl
from jax.experimental.pallas import tpu as pltpu
from jax.experimental.pallas import tpu_sc as plsc

NREAL = 10000
N = 10240
E = 320000
F = 128
NC = 2
NS = 16
NW = NC * NS
B = 48
STEPS = 216
EPT = B * STEPS
E_PAD = NW * EPT
NBUF = 4
RPW = N // NS

_vmesh = plsc.VectorSubcoreMesh(core_axis_name="c", subcore_axis_name="s")


NH = N // 128


def _deg_body(dsti_hbm, z_hbm, iota_hbm, out_hbm, idx_v, hist_v, iota_v,
              acc_sh):
    cid = lax.axis_index("c")
    sid = lax.axis_index("s")
    wid = cid * NS + sid

    pltpu.sync_copy(z_hbm.at[pl.ds(0, NH)], hist_v)
    pltpu.sync_copy(iota_hbm, iota_v)
    pltpu.sync_copy(dsti_hbm.at[wid], idx_v)

    @pl.when(sid == 0)
    def _():
        pltpu.sync_copy(z_hbm.at[pl.ds(0, NH)], acc_sh)

    ones16 = jnp.full((16,), 1.0, jnp.float32)

    @pl.loop(0, EPT, step=16)
    def _(k):
        iv = idx_v[pl.ds(k, 16)]
        r = lax.shift_right_logical(iv, 7)
        c = lax.bitwise_and(iv, 127)
        plsc.addupdate_scatter(hist_v, [r, c], ones16)

    plsc.subcore_barrier()
    pltpu.sync_copy(hist_v, acc_sh.at[iota_v], add=True)
    plsc.subcore_barrier()

    @pl.when(sid == 0)
    def _():
        pltpu.sync_copy(acc_sh, out_hbm.at[pl.ds(cid * NH, NH)])


_cp_sc = pltpu.CompilerParams()
if "needs_layout_passes" in pltpu.CompilerParams.__dataclass_fields__:
    _cp_sc = dataclasses.replace(_cp_sc, needs_layout_passes=False)

_deg_call = functools.partial(
    pl.kernel,
    compiler_params=_cp_sc,
    out_type=jax.ShapeDtypeStruct((NC * NH, 128), jnp.float32),
    mesh=_vmesh,
    scratch_types=[
        pltpu.VMEM((EPT,), jnp.int32),
        pltpu.VMEM((NH, 128), jnp.float32),
        pltpu.VMEM((NH,), jnp.int32),
        pltpu.VMEM_SHARED((NH, 128), jnp.float32),
    ],
)(_deg_body)


def _agg_body(srci_hbm, dsti_hbm, m_hbm, z_hbm, out_hbm,
              idxs_v, idxd_v, r0, r1, r2, r3, acc_sh,
              g0, g1, g2, g3, s0, s1, s2, s3):
    cid = lax.axis_index("c")
    sid = lax.axis_index("s")
    wid = cid * NS + sid
    rows = [r0, r1, r2, r3]
    gsem = [g0, g1, g2, g3]
    ssem = [s0, s1, s2, s3]

    pltpu.sync_copy(z_hbm.at[pl.ds(sid * RPW, RPW)],
                    acc_sh.at[pl.ds(sid * RPW, RPW)])
    pltpu.sync_copy(srci_hbm.at[wid], idxs_v)
    pltpu.sync_copy(dsti_hbm.at[wid], idxd_v)
    plsc.subcore_barrier()

    for b in range(NBUF - 1):
        pltpu.async_copy(m_hbm.at[idxs_v.at[pl.ds(b * B, B)]], rows[b],
                         gsem[b])

    @pl.loop(0, STEPS, step=NBUF)
    def _(g):
        for b in range(NBUF):
            t = g + b
            bg = (b + NBUF - 1) % NBUF

            pltpu.make_async_copy(
                m_hbm.at[idxs_v.at[pl.ds(t * B, B)]], rows[b],
                gsem[b]).wait()
            pltpu.async_copy(rows[b], acc_sh.at[idxd_v.at[pl.ds(t * B, B)]],
                             ssem[b], add=True)

            @pl.when(t >= 1)
            def _():
                pltpu.make_async_copy(
                    rows[bg], acc_sh.at[idxd_v.at[pl.ds((t - 1) * B, B)]],
                    ssem[bg]).wait()

            @pl.when(t + NBUF - 1 < STEPS)
            def _():
                pltpu.async_copy(
                    m_hbm.at[idxs_v.at[pl.ds((t + NBUF - 1) * B, B)]],
                    rows[bg], gsem[bg])

    pltpu.make_async_copy(
        rows[(STEPS - 1) % NBUF],
        acc_sh.at[idxd_v.at[pl.ds((STEPS - 1) * B, B)]],
        ssem[(STEPS - 1) % NBUF]).wait()

    plsc.subcore_barrier()
    pltpu.sync_copy(acc_sh.at[pl.ds(sid * RPW, RPW)],
                    out_hbm.at[pl.ds(cid * N + sid * RPW, RPW)])


_agg_call = functools.partial(
    pl.kernel,
    out_type=jax.ShapeDtypeStruct((NC * N, F), jnp.float32),
    mesh=_vmesh,
    scratch_types=[
        pltpu.VMEM((EPT,), jnp.int32),
        pltpu.VMEM((EPT,), jnp.int32),
        pltpu.VMEM((B, F), jnp.float32),
        pltpu.VMEM((B, F), jnp.float32),
        pltpu.VMEM((B, F), jnp.float32),
        pltpu.VMEM((B, F), jnp.float32),
        pltpu.VMEM_SHARED((N, F), jnp.float32),
        pltpu.SemaphoreType.DMA,
        pltpu.SemaphoreType.DMA,
        pltpu.SemaphoreType.DMA,
        pltpu.SemaphoreType.DMA,
        pltpu.SemaphoreType.DMA,
        pltpu.SemaphoreType.DMA,
        pltpu.SemaphoreType.DMA,
        pltpu.SemaphoreType.DMA,
    ],
)(_agg_body)


_RB = 1024
_GRID = N // _RB


def _mm_body(x_ref, w_ref, o_ref):
    o_ref[...] = jnp.dot(x_ref[...], w_ref[...],
                         preferred_element_type=jnp.float32)


def _tc_matmul(x, w):
    return pl.pallas_call(
        _mm_body,
        grid=(_GRID,),
        in_specs=[pl.BlockSpec((_RB, F), lambda i: (i, 0)),
                  pl.BlockSpec((F, F), lambda i: (0, 0))],
        out_specs=pl.BlockSpec((_RB, F), lambda i: (i, 0)),
        out_shape=jax.ShapeDtypeStruct((N, F), jnp.float32),
    )(x, w)


def _scale_body(degp_ref, xw_ref, xs_ref, dis_ref):
    deg = degp_ref[0] + degp_ref[1] + 1.0
    dis = lax.rsqrt(deg)
    dis_ref[...] = dis
    xs_ref[...] = xw_ref[...] * dis


def _tc_scale(degp, xw):
    return pl.pallas_call(
        _scale_body,
        grid=(_GRID,),
        in_specs=[pl.BlockSpec((NC, _RB, 1), lambda i: (0, i, 0)),
                  pl.BlockSpec((_RB, F), lambda i: (i, 0))],
        out_specs=[pl.BlockSpec((_RB, F), lambda i: (i, 0)),
                   pl.BlockSpec((_RB, 1), lambda i: (i, 0))],
        out_shape=[jax.ShapeDtypeStruct((N, F), jnp.float32),
                   jax.ShapeDtypeStruct((N, 1), jnp.float32)],
    )(degp, xw)


def _relu_body(tp_ref, xs_ref, dis_ref, b_ref, hs_ref):
    dis = dis_ref[...]
    z = (tp_ref[0] + tp_ref[1] + xs_ref[...]) * dis + b_ref[...]
    hs_ref[...] = jnp.maximum(z, 0.0) * dis


def _tc_relu_scale(tp, xs, dis, b):
    return pl.pallas_call(
        _relu_body,
        grid=(_GRID,),
        in_specs=[pl.BlockSpec((NC, _RB, F), lambda i: (0, i, 0)),
                  pl.BlockSpec((_RB, F), lambda i: (i, 0)),
                  pl.BlockSpec((_RB, 1), lambda i: (i, 0)),
                  pl.BlockSpec((1, F), lambda i: (0, 0))],
        out_specs=pl.BlockSpec((_RB, F), lambda i: (i, 0)),
        out_shape=jax.ShapeDtypeStruct((N, F), jnp.float32),
    )(tp, xs, dis, b)


def _head_body(tp_ref, hs_ref, dis_ref, w_ref, b_ref, o_ref):
    p = (tp_ref[0] + tp_ref[1] + hs_ref[...]) * dis_ref[...]
    o_ref[...] = jnp.dot(p, w_ref[...],
                         preferred_element_type=jnp.float32) + b_ref[...]


def _tc_heads(tp, hs, dis, w, b):
    return pl.pallas_call(
        _head_body,
        grid=(_GRID,),
        in_specs=[pl.BlockSpec((NC, _RB, F), lambda i: (0, i, 0)),
                  pl.BlockSpec((_RB, F), lambda i: (i, 0)),
                  pl.BlockSpec((_RB, 1), lambda i: (i, 0)),
                  pl.BlockSpec((F, F), lambda i: (0, 0)),
                  pl.BlockSpec((1, F), lambda i: (0, 0))],
        out_specs=pl.BlockSpec((_RB, F), lambda i: (i, 0)),
        out_shape=jax.ShapeDtypeStruct((N, F), jnp.float32),
    )(tp, hs, dis, w, b)


def kernel(X, edge_index, W1, b1, Wmu, bmu, Wlv, blv):
    pad_idx = NREAL + jnp.arange(E_PAD - E, dtype=jnp.int32) % (N - NREAL)
    ei = jnp.concatenate([edge_index, jnp.stack([pad_idx, pad_idx])], axis=1)
    srci = ei[0].reshape(NW, EPT)
    dsti = ei[1].reshape(NW, EPT)
    Xp = jnp.pad(X, ((0, N - NREAL), (0, 0)))
    z2 = jnp.zeros((N, F), jnp.float32)
    iota = jnp.arange(NH, dtype=jnp.int32)

    degp = _deg_call(dsti, z2, iota)
    xw = _tc_matmul(Xp, W1)
    xs, dis = _tc_scale(degp.reshape(NC, N, 1), xw)

    t1 = _agg_call(srci, dsti, xs, z2)
    hs = _tc_relu_scale(t1.reshape(NC, N, F), xs, dis, b1.reshape(1, F))

    t2 = _agg_call(srci, dsti, hs, z2)
    wcat = jnp.concatenate([Wmu, Wlv], axis=1)
    bcat = jnp.concatenate([bmu, blv]).reshape(1, F)
    out = _tc_heads(t2.reshape(NC, N, F), hs, dis, wcat, bcat)

    return out[:NREAL, :64], out[:NREAL, 64:]


# speedup vs baseline: 1.0226x; 1.0226x over previous
import dataclasses
import functools

import jax
import jax.numpy as jnp
from jax import lax
from jax.experimental import pallas as pl
from jax.experimental.pallas import tpu as pltpu
from jax.experimental.pallas import tpu_sc as plsc

NREAL = 10000
N = 10240
E = 320000
F = 128
NC = 2
NS = 16
NW = NC * NS
B = 72
STEPS = 144
EPT = B * STEPS
E_PAD = NW * EPT
NBUF = 3
RPW = N // NS

_vmesh = plsc.VectorSubcoreMesh(core_axis_name="c", subcore_axis_name="s")


NH = N // 128


def _deg_body(dsti_hbm, z_hbm, iota_hbm, out_hbm, idx_v, hist_v, iota_v,
              acc_sh):
    cid = lax.axis_index("c")
    sid = lax.axis_index("s")
    wid = cid * NS + sid

    pltpu.sync_copy(z_hbm.at[pl.ds(0, NH)], hist_v)
    pltpu.sync_copy(iota_hbm, iota_v)
    pltpu.sync_copy(dsti_hbm.at[wid], idx_v)

    @pl.when(sid == 0)
    def _():
        pltpu.sync_copy(z_hbm.at[pl.ds(0, NH)], acc_sh)

    ones16 = jnp.full((16,), 1.0, jnp.float32)

    @pl.loop(0, EPT, step=16)
    def _(k):
        iv = idx_v[pl.ds(k, 16)]
        r = lax.shift_right_logical(iv, 7)
        c = lax.bitwise_and(iv, 127)
        plsc.addupdate_scatter(hist_v, [r, c], ones16)

    plsc.subcore_barrier()
    pltpu.sync_copy(hist_v, acc_sh.at[iota_v], add=True)
    plsc.subcore_barrier()

    @pl.when(sid == 0)
    def _():
        pltpu.sync_copy(acc_sh, out_hbm.at[pl.ds(cid * NH, NH)])


_cp_sc = pltpu.CompilerParams()
if "needs_layout_passes" in pltpu.CompilerParams.__dataclass_fields__:
    _cp_sc = dataclasses.replace(_cp_sc, needs_layout_passes=False)

_deg_call = functools.partial(
    pl.kernel,
    compiler_params=_cp_sc,
    out_type=jax.ShapeDtypeStruct((NC * NH, 128), jnp.float32),
    mesh=_vmesh,
    scratch_types=[
        pltpu.VMEM((EPT,), jnp.int32),
        pltpu.VMEM((NH, 128), jnp.float32),
        pltpu.VMEM((NH,), jnp.int32),
        pltpu.VMEM_SHARED((NH, 128), jnp.float32),
    ],
)(_deg_body)


def _agg_body(srci_hbm, dsti_hbm, m_hbm, z_hbm, out_hbm,
              idxs_v, idxd_v, r0, r1, r2, acc_sh,
              g0, g1, g2, s0, s1, s2):
    cid = lax.axis_index("c")
    sid = lax.axis_index("s")
    wid = cid * NS + sid
    rows = [r0, r1, r2]
    gsem = [g0, g1, g2]
    ssem = [s0, s1, s2]

    pltpu.sync_copy(z_hbm.at[pl.ds(sid * RPW, RPW)],
                    acc_sh.at[pl.ds(sid * RPW, RPW)])
    pltpu.sync_copy(srci_hbm.at[wid], idxs_v)
    pltpu.sync_copy(dsti_hbm.at[wid], idxd_v)
    plsc.subcore_barrier()

    for b in range(NBUF - 1):
        pltpu.async_copy(m_hbm.at[idxs_v.at[pl.ds(b * B, B)]], rows[b],
                         gsem[b])

    @pl.loop(0, STEPS, step=NBUF)
    def _(g):
        for b in range(NBUF):
            t = g + b
            bg = (b + NBUF - 1) % NBUF

            pltpu.make_async_copy(
                m_hbm.at[idxs_v.at[pl.ds(t * B, B)]], rows[b],
                gsem[b]).wait()
            pltpu.async_copy(rows[b], acc_sh.at[idxd_v.at[pl.ds(t * B, B)]],
                             ssem[b], add=True)

            @pl.when(t >= 1)
            def _():
                pltpu.make_async_copy(
                    rows[bg], acc_sh.at[idxd_v.at[pl.ds((t - 1) * B, B)]],
                    ssem[bg]).wait()

            @pl.when(t + NBUF - 1 < STEPS)
            def _():
                pltpu.async_copy(
                    m_hbm.at[idxs_v.at[pl.ds((t + NBUF - 1) * B, B)]],
                    rows[bg], gsem[bg])

    pltpu.make_async_copy(
        rows[(STEPS - 1) % NBUF],
        acc_sh.at[idxd_v.at[pl.ds((STEPS - 1) * B, B)]],
        ssem[(STEPS - 1) % NBUF]).wait()

    plsc.subcore_barrier()
    pltpu.sync_copy(acc_sh.at[pl.ds(sid * RPW, RPW)],
                    out_hbm.at[pl.ds(cid * N + sid * RPW, RPW)])


_agg_call = functools.partial(
    pl.kernel,
    out_type=jax.ShapeDtypeStruct((NC * N, F), jnp.float32),
    mesh=_vmesh,
    scratch_types=[
        pltpu.VMEM((EPT,), jnp.int32),
        pltpu.VMEM((EPT,), jnp.int32),
        pltpu.VMEM((B, F), jnp.float32),
        pltpu.VMEM((B, F), jnp.float32),
        pltpu.VMEM((B, F), jnp.float32),
        pltpu.VMEM_SHARED((N, F), jnp.float32),
        pltpu.SemaphoreType.DMA,
        pltpu.SemaphoreType.DMA,
        pltpu.SemaphoreType.DMA,
        pltpu.SemaphoreType.DMA,
        pltpu.SemaphoreType.DMA,
        pltpu.SemaphoreType.DMA,
    ],
)(_agg_body)


_RB = 1024
_GRID = N // _RB


def _mm_scale_body(degp_ref, x_ref, w_ref, xs_ref, dis_ref):
    deg = degp_ref[0] + degp_ref[1] + 1.0
    dis = lax.rsqrt(deg)
    dis_ref[...] = dis
    xw = jnp.dot(x_ref[...], w_ref[...], preferred_element_type=jnp.float32)
    xs_ref[...] = xw * dis


def _tc_mm_scale(degp, x, w):
    return pl.pallas_call(
        _mm_scale_body,
        grid=(_GRID,),
        in_specs=[pl.BlockSpec((NC, _RB, 1), lambda i: (0, i, 0)),
                  pl.BlockSpec((_RB, F), lambda i: (i, 0)),
                  pl.BlockSpec((F, F), lambda i: (0, 0))],
        out_specs=[pl.BlockSpec((_RB, F), lambda i: (i, 0)),
                   pl.BlockSpec((_RB, 1), lambda i: (i, 0))],
        out_shape=[jax.ShapeDtypeStruct((N, F), jnp.float32),
                   jax.ShapeDtypeStruct((N, 1), jnp.float32)],
    )(degp, x, w)


def _relu_body(tp_ref, xs_ref, dis_ref, b_ref, hs_ref):
    dis = dis_ref[...]
    z = (tp_ref[0] + tp_ref[1] + xs_ref[...]) * dis + b_ref[...]
    hs_ref[...] = jnp.maximum(z, 0.0) * dis


def _tc_relu_scale(tp, xs, dis, b):
    return pl.pallas_call(
        _relu_body,
        grid=(_GRID,),
        in_specs=[pl.BlockSpec((NC, _RB, F), lambda i: (0, i, 0)),
                  pl.BlockSpec((_RB, F), lambda i: (i, 0)),
                  pl.BlockSpec((_RB, 1), lambda i: (i, 0)),
                  pl.BlockSpec((1, F), lambda i: (0, 0))],
        out_specs=pl.BlockSpec((_RB, F), lambda i: (i, 0)),
        out_shape=jax.ShapeDtypeStruct((N, F), jnp.float32),
    )(tp, xs, dis, b)


def _head_body(tp_ref, hs_ref, dis_ref, w_ref, b_ref, o_ref):
    p = (tp_ref[0] + tp_ref[1] + hs_ref[...]) * dis_ref[...]
    o_ref[...] = jnp.dot(p, w_ref[...],
                         preferred_element_type=jnp.float32) + b_ref[...]


def _tc_heads(tp, hs, dis, w, b):
    return pl.pallas_call(
        _head_body,
        grid=(_GRID,),
        in_specs=[pl.BlockSpec((NC, _RB, F), lambda i: (0, i, 0)),
                  pl.BlockSpec((_RB, F), lambda i: (i, 0)),
                  pl.BlockSpec((_RB, 1), lambda i: (i, 0)),
                  pl.BlockSpec((F, F), lambda i: (0, 0)),
                  pl.BlockSpec((1, F), lambda i: (0, 0))],
        out_specs=pl.BlockSpec((_RB, F), lambda i: (i, 0)),
        out_shape=jax.ShapeDtypeStruct((N, F), jnp.float32),
    )(tp, hs, dis, w, b)


def kernel(X, edge_index, W1, b1, Wmu, bmu, Wlv, blv):
    pad_idx = NREAL + jnp.arange(E_PAD - E, dtype=jnp.int32) % (N - NREAL)
    ei = jnp.concatenate([edge_index, jnp.stack([pad_idx, pad_idx])], axis=1)
    srci = ei[0].reshape(NW, EPT)
    dsti = ei[1].reshape(NW, EPT)
    Xp = jnp.pad(X, ((0, N - NREAL), (0, 0)))
    z2 = jnp.zeros((N, F), jnp.float32)
    iota = jnp.arange(NH, dtype=jnp.int32)

    degp = _deg_call(dsti, z2, iota)
    xs, dis = _tc_mm_scale(degp.reshape(NC, N, 1), Xp, W1)

    t1 = _agg_call(srci, dsti, xs, z2)
    hs = _tc_relu_scale(t1.reshape(NC, N, F), xs, dis, b1.reshape(1, F))

    t2 = _agg_call(srci, dsti, hs, z2)
    wcat = jnp.concatenate([Wmu, Wlv], axis=1)
    bcat = jnp.concatenate([bmu, blv]).reshape(1, F)
    out = _tc_heads(t2.reshape(NC, N, F), hs, dis, wcat, bcat)

    return out[:NREAL, :64], out[:NREAL, 64:]
